# baseline (speedup 1.0000x reference)
import functools

import jax
import jax.numpy as jnp
from jax.experimental import pallas as pl

LEVELS = 4
K = 1024
LATENT = 64
BLOCK_B = 512


def _rqvae_block(h1_ref, We1, be1, We2, be2,
                 cbs_ref, cbt_ref, cbn_ref,
                 Wd0, bd0, Wd1, bd1, Wd2, bd2,
                 dec_ref, r_ref, e_ref, cnt_ref, q_ref):
    f32 = jnp.float32
    bf16 = jnp.bfloat16

    def dot16(a, b):
        return jnp.dot(a.astype(bf16), b.astype(bf16), preferred_element_type=f32)

    h = h1_ref[...]
    h = jnp.maximum(dot16(h, We1[...]) + be1[...], 0.0)
    z = dot16(h, We2[...]) + be2[...]

    bs = z.shape[0]
    iota_k = jax.lax.broadcasted_iota(jnp.int32, (bs, K), 1)
    ones_row = jnp.ones((1, bs), dtype=bf16)

    residual = z
    z_hat = jnp.zeros_like(z)
    idx_cols, cnt_rows = [], []
    for l in range(LEVELS):
        v = residual * residual
        s = v[:, 0:8]
        for g in range(1, 8):
            s = s + v[:, 8 * g:8 * (g + 1)]
        s = s[:, 0:4] + s[:, 4:8]
        s = s[:, 0:2] + s[:, 2:4]
        rnorm = s[:, 0:1] + s[:, 1:2]
        d = (rnorm + dot16(residual, cbt_ref[l])) + cbn_ref[l]
        m = jnp.min(d, axis=1, keepdims=True)
        idx = jnp.min(jnp.where(d == m, iota_k, K), axis=1, keepdims=True)
        onehot = (iota_k == idx).astype(bf16)
        parts = jnp.dot(onehot, cbs_ref[l], preferred_element_type=f32)
        e_l = (parts[:, :LATENT] + parts[:, LATENT:2 * LATENT]) \
              + parts[:, 2 * LATENT:]
        cnt_rows.append(jnp.dot(ones_row, onehot, preferred_element_type=f32))
        idx_cols.append(idx)
        r_ref[l] = residual
        e_ref[l] = e_l
        z_hat = z_hat + e_l
        residual = residual - e_l

    q_ref[...] = jnp.concatenate(idx_cols, axis=1)
    stacked = jnp.concatenate(cnt_rows, axis=0).astype(jnp.int32)

    @pl.when(pl.program_id(0) == 0)
    def _():
        cnt_ref[...] = stacked

    @pl.when(pl.program_id(0) != 0)
    def _():
        cnt_ref[...] = cnt_ref[...] + stacked

    zst = z + (z_hat - z)
    g = jnp.maximum(dot16(zst, Wd0[...]) + bd0[...], 0.0)
    g = jnp.maximum(dot16(g, Wd1[...]) + bd1[...], 0.0)
    dec_ref[...] = dot16(g, Wd2[...]) + bd2[...]


@functools.partial(jax.jit, static_argnames=())
def kernel(x, We0, be0, We1, be1, We2, be2, codebooks, Wd0, bd0, Wd1, bd1, Wd2, bd2):
    B, IN = x.shape
    bs = BLOCK_B
    grid = (B // bs,)
    h1 = jax.nn.relu(x @ We0 + be0)
    f32 = jnp.float32
    bf16 = jnp.bfloat16
    cbt = (-2.0 * codebooks).transpose(0, 2, 1)
    cbn = jnp.sum(codebooks * codebooks, axis=2)[:, None, :]
    def _trunc16(a):
        u = jax.lax.bitcast_convert_type(a, jnp.uint32)
        return jax.lax.bitcast_convert_type(u & jnp.uint32(0xFFFF0000), f32)

    cb_hi = _trunc16(codebooks)
    r1 = codebooks - cb_hi
    cb_mid = _trunc16(r1)
    cb_lo = r1 - cb_mid
    cb_split = jnp.concatenate(
        [cb_hi.astype(bf16), cb_mid.astype(bf16), cb_lo.astype(bf16)],
        axis=2)
    OUT = Wd2.shape[1]

    def full(a):
        return pl.BlockSpec(a.shape, lambda i: (0,) * a.ndim)

    b2 = [b.reshape(1, -1) for b in (be1, be2, bd0, bd1, bd2)]
    H1 = We0.shape[1]

    out_shapes = (
        jax.ShapeDtypeStruct((B, OUT), jnp.float32),
        jax.ShapeDtypeStruct((LEVELS, B, LATENT), jnp.float32),
        jax.ShapeDtypeStruct((LEVELS, B, LATENT), jnp.float32),
        jax.ShapeDtypeStruct((LEVELS, K), jnp.int32),
        jax.ShapeDtypeStruct((B, LEVELS), jnp.int32),
    )
    out_specs = (
        pl.BlockSpec((bs, OUT), lambda i: (i, 0)),
        pl.BlockSpec((LEVELS, bs, LATENT), lambda i: (0, i, 0)),
        pl.BlockSpec((LEVELS, bs, LATENT), lambda i: (0, i, 0)),
        pl.BlockSpec((LEVELS, K), lambda i: (0, 0)),
        pl.BlockSpec((bs, LEVELS), lambda i: (i, 0)),
    )
    in_specs = [
        pl.BlockSpec((bs, H1), lambda i: (i, 0)),
        full(We1), full(b2[0]), full(We2), full(b2[1]),
        full(cb_split), full(cbt), full(cbn),
        full(Wd0), full(b2[2]), full(Wd1), full(b2[3]), full(Wd2), full(b2[4]),
    ]

    decoded, r, e, counts, quantized = pl.pallas_call(
        _rqvae_block,
        grid=grid,
        in_specs=in_specs,
        out_specs=out_specs,
        out_shape=out_shapes,
    )(h1, We1, b2[0], We2, b2[1], cb_split, cbt, cbn,
      Wd0, b2[2], Wd1, b2[3], Wd2, b2[4])
    return (decoded, r, e, counts, quantized)


# speedup vs baseline: 1.0015x; 1.0015x over previous
import functools

import jax
import jax.numpy as jnp
from jax.experimental import pallas as pl

LEVELS = 4
K = 1024
LATENT = 64
BLOCK_B = 1024


def _rqvae_block(h1_ref, We1, be1, We2, be2,
                 cbs_ref, cbt_ref, cbn_ref,
                 Wd0, bd0, Wd1, bd1, Wd2, bd2,
                 dec_ref, r_ref, e_ref, cnt_ref, q_ref):
    f32 = jnp.float32
    bf16 = jnp.bfloat16

    def dot16(a, b):
        return jnp.dot(a.astype(bf16), b.astype(bf16), preferred_element_type=f32)

    h = h1_ref[...]
    h = jnp.maximum(dot16(h, We1[...]) + be1[...], 0.0)
    z = dot16(h, We2[...]) + be2[...]

    bs = z.shape[0]
    iota_k = jax.lax.broadcasted_iota(jnp.int32, (bs, K), 1)
    ones_row = jnp.ones((1, bs), dtype=bf16)

    residual = z
    z_hat = jnp.zeros_like(z)
    idx_cols, cnt_rows = [], []
    for l in range(LEVELS):
        v = residual * residual
        s = v[:, 0:8]
        for g in range(1, 8):
            s = s + v[:, 8 * g:8 * (g + 1)]
        s = s[:, 0:4] + s[:, 4:8]
        s = s[:, 0:2] + s[:, 2:4]
        rnorm = s[:, 0:1] + s[:, 1:2]
        d = (rnorm + dot16(residual, cbt_ref[l])) + cbn_ref[l]
        m = jnp.min(d, axis=1, keepdims=True)
        idx = jnp.min(jnp.where(d == m, iota_k, K), axis=1, keepdims=True)
        onehot = (iota_k == idx).astype(bf16)
        parts = jnp.dot(onehot, cbs_ref[l], preferred_element_type=f32)
        e_l = (parts[:, :LATENT] + parts[:, LATENT:2 * LATENT]) \
              + parts[:, 2 * LATENT:]
        cnt_rows.append(jnp.dot(ones_row, onehot, preferred_element_type=f32))
        idx_cols.append(idx)
        r_ref[l] = residual
        e_ref[l] = e_l
        z_hat = z_hat + e_l
        residual = residual - e_l

    q_ref[...] = jnp.concatenate(idx_cols, axis=1)
    stacked = jnp.concatenate(cnt_rows, axis=0).astype(jnp.int32)

    @pl.when(pl.program_id(0) == 0)
    def _():
        cnt_ref[...] = stacked

    @pl.when(pl.program_id(0) != 0)
    def _():
        cnt_ref[...] = cnt_ref[...] + stacked

    zst = z + (z_hat - z)
    g = jnp.maximum(dot16(zst, Wd0[...]) + bd0[...], 0.0)
    g = jnp.maximum(dot16(g, Wd1[...]) + bd1[...], 0.0)
    dec_ref[...] = dot16(g, Wd2[...]) + bd2[...]


@functools.partial(jax.jit, static_argnames=())
def kernel(x, We0, be0, We1, be1, We2, be2, codebooks, Wd0, bd0, Wd1, bd1, Wd2, bd2):
    B, IN = x.shape
    bs = BLOCK_B
    grid = (B // bs,)
    h1 = jax.nn.relu(x @ We0 + be0)
    f32 = jnp.float32
    bf16 = jnp.bfloat16
    cbt = (-2.0 * codebooks).transpose(0, 2, 1)
    cbn = jnp.sum(codebooks * codebooks, axis=2)[:, None, :]
    def _trunc16(a):
        u = jax.lax.bitcast_convert_type(a, jnp.uint32)
        return jax.lax.bitcast_convert_type(u & jnp.uint32(0xFFFF0000), f32)

    cb_hi = _trunc16(codebooks)
    r1 = codebooks - cb_hi
    cb_mid = _trunc16(r1)
    cb_lo = r1 - cb_mid
    cb_split = jnp.concatenate(
        [cb_hi.astype(bf16), cb_mid.astype(bf16), cb_lo.astype(bf16)],
        axis=2)
    OUT = Wd2.shape[1]

    def full(a):
        return pl.BlockSpec(a.shape, lambda i: (0,) * a.ndim)

    b2 = [b.reshape(1, -1) for b in (be1, be2, bd0, bd1, bd2)]
    H1 = We0.shape[1]

    out_shapes = (
        jax.ShapeDtypeStruct((B, OUT), jnp.float32),
        jax.ShapeDtypeStruct((LEVELS, B, LATENT), jnp.float32),
        jax.ShapeDtypeStruct((LEVELS, B, LATENT), jnp.float32),
        jax.ShapeDtypeStruct((LEVELS, K), jnp.int32),
        jax.ShapeDtypeStruct((B, LEVELS), jnp.int32),
    )
    out_specs = (
        pl.BlockSpec((bs, OUT), lambda i: (i, 0)),
        pl.BlockSpec((LEVELS, bs, LATENT), lambda i: (0, i, 0)),
        pl.BlockSpec((LEVELS, bs, LATENT), lambda i: (0, i, 0)),
        pl.BlockSpec((LEVELS, K), lambda i: (0, 0)),
        pl.BlockSpec((bs, LEVELS), lambda i: (i, 0)),
    )
    in_specs = [
        pl.BlockSpec((bs, H1), lambda i: (i, 0)),
        full(We1), full(b2[0]), full(We2), full(b2[1]),
        full(cb_split), full(cbt), full(cbn),
        full(Wd0), full(b2[2]), full(Wd1), full(b2[3]), full(Wd2), full(b2[4]),
    ]

    decoded, r, e, counts, quantized = pl.pallas_call(
        _rqvae_block,
        grid=grid,
        in_specs=in_specs,
        out_specs=out_specs,
        out_shape=out_shapes,
    )(h1, We1, b2[0], We2, b2[1], cb_split, cbt, cbn,
      Wd0, b2[2], Wd1, b2[3], Wd2, b2[4])
    return (decoded, r, e, counts, quantized)
